# Initial kernel scaffold; baseline (speedup 1.0000x reference)
import jax
import jax.numpy as jnp
from jax.experimental import pallas as pl


def kernel(x, edge_index_sc, edge_index_fc, batch, Wl0, bl0, Wr0, g0, b0, Wl1, bl1, Wr1, g1, b1, Wl2, bl2, Wr2):
    raise NotImplementedError("write your pallas kernel here")


# speedup vs baseline: 2.7680x; 2.7680x over previous
import jax
import jax.numpy as jnp
from jax import lax
from jax.experimental import pallas as pl
from jax.experimental.pallas import tpu as pltpu
from jax.experimental.pallas import tpu_sc as plsc

N = 10000
E = 160000
D_IN = 256
D_H = 512
G = 64

CW = 128
NCX = D_IN // CW
NCH = D_H // CW
NT = 16
K = 50
NCHUNK = 200
NGRP = 5
GCH = NCHUNK // NGRP
RPT = 624
RTAIL = N - NT * RPT
R = 1000
NB = N // R

_mesh = plsc.VectorSubcoreMesh(core_axis_name="c", subcore_axis_name="s")
_f32 = jnp.float32


def _part_copy(src, dst, sid, cond=None):
    sl = pl.ds(sid * RPT, RPT)
    tl = pl.ds(NT * RPT, RTAIL)
    is_last = sid == NT - 1
    if cond is None:
        pltpu.sync_copy(src.at[sl], dst.at[sl])

        @pl.when(is_last)
        def _():
            pltpu.sync_copy(src.at[tl], dst.at[tl])
    else:
        @pl.when(cond)
        def _():
            pltpu.sync_copy(src.at[sl], dst.at[sl])

        @pl.when(cond & is_last)
        def _():
            pltpu.sync_copy(src.at[tl], dst.at[tl])


def _sc_cnt_body(dst0, dst1, z, ones_h, cnt0, cnt1,
                 dst_v, ones_v, acc):
    cid = lax.axis_index("c")
    sid = lax.axis_index("s")
    _part_copy(z, acc, sid)
    pltpu.sync_copy(ones_h, ones_v)
    plsc.subcore_barrier()

    def loop(j, carry):
        pltpu.sync_copy(ones_v, acc.at[dst_v.at[j]], add=True)
        return carry

    for g in range(NGRP):
        @pl.when(cid == 0)
        def _():
            pltpu.sync_copy(dst0.at[sid, pl.ds(g * GCH, GCH)], dst_v)

        @pl.when(cid == 1)
        def _():
            pltpu.sync_copy(dst1.at[sid, pl.ds(g * GCH, GCH)], dst_v)

        lax.fori_loop(0, GCH, loop, 0)

    plsc.subcore_barrier()
    _part_copy(acc, cnt0, sid, cond=cid == 0)
    _part_copy(acc, cnt1, sid, cond=cid == 1)


_sc_cnt = pl.kernel(
    _sc_cnt_body,
    out_type=(jax.ShapeDtypeStruct((N, CW), _f32),
              jax.ShapeDtypeStruct((N, CW), _f32)),
    mesh=_mesh,
    scratch_types=[
        pltpu.VMEM((GCH, K), jnp.int32),
        pltpu.VMEM((K, CW), _f32),
        pltpu.VMEM_SHARED((N, CW), _f32),
    ],
)


def _make_sc_seg(nchunks):
    npass = nchunks // 2

    def body(*refs):
        pos = 0
        hs = refs[pos:pos + nchunks]; pos += nchunks
        srcr, dstr, z = refs[pos:pos + 3]; pos += 3
        outs = refs[pos:pos + nchunks]; pos += nchunks
        src_v, dst_v, rows, sem, acc = refs[pos:pos + 5]

        cid = lax.axis_index("c")
        sid = lax.axis_index("s")

        for p in range(npass):
            h_sc0 = hs[p]
            h_sc1 = hs[npass + p]

            _part_copy(z, acc, sid)
            plsc.subcore_barrier()

            def loop_sc0(j, carry):
                pltpu.async_copy(h_sc0.at[src_v.at[j]], rows, sem).wait()
                pltpu.sync_copy(rows, acc.at[dst_v.at[j]], add=True)
                return carry

            def loop_sc1(j, carry):
                pltpu.async_copy(h_sc1.at[src_v.at[j]], rows, sem).wait()
                pltpu.sync_copy(rows, acc.at[dst_v.at[j]], add=True)
                return carry

            for g in range(NGRP):
                pltpu.sync_copy(srcr.at[sid, pl.ds(g * GCH, GCH)], src_v)
                pltpu.sync_copy(dstr.at[sid, pl.ds(g * GCH, GCH)], dst_v)

                @pl.when(cid == 0)
                def _():
                    lax.fori_loop(0, GCH, loop_sc0, 0)

                @pl.when(cid == 1)
                def _():
                    lax.fori_loop(0, GCH, loop_sc1, 0)

            plsc.subcore_barrier()
            _part_copy(acc, outs[p], sid, cond=cid == 0)
            _part_copy(acc, outs[npass + p], sid, cond=cid == 1)
            plsc.subcore_barrier()

    out_type = [jax.ShapeDtypeStruct((N, CW), _f32) for _ in range(nchunks)]
    scratch = [
        pltpu.VMEM((GCH, K), jnp.int32),
        pltpu.VMEM((GCH, K), jnp.int32),
        pltpu.VMEM((K, CW), _f32),
        pltpu.SemaphoreType.DMA,
        pltpu.VMEM_SHARED((N, CW), _f32),
    ]
    return pl.kernel(body, out_type=tuple(out_type), mesh=_mesh,
                     scratch_types=scratch)


_sc_l0 = _make_sc_seg(NCX)
_sc_agg = _make_sc_seg(NCH)


def _stats_update(st_ref, b, y, i):
    st = jnp.concatenate(
        [jnp.sum(y, axis=0, keepdims=True),
         jnp.sum(y * y, axis=0, keepdims=True),
         jnp.zeros((6, D_H), _f32)], axis=0)

    @pl.when(i == 0)
    def _():
        st_ref[b] = st

    @pl.when(i != 0)
    def _():
        st_ref[b] += st


def _dense_l0_body(*refs):
    x_ref = refs[0]
    aggs = refs[1:1 + 2 * NCX]
    c0_ref, c1_ref, wl_ref, wr_ref, bl_ref = refs[1 + 2 * NCX:6 + 2 * NCX]
    y0_ref, y1_ref, st_ref = refs[6 + 2 * NCX:]
    i = pl.program_id(0)
    xw = jnp.dot(x_ref[...], wr_ref[...], preferred_element_type=_f32)
    for b in range(2):
        ac = aggs[b * NCX:(b + 1) * NCX]
        cref = c0_ref if b == 0 else c1_ref
        inv = 1.0 / jnp.maximum(cref[0], 1.0)
        agg = jnp.concatenate([a[...] for a in ac], axis=1) * inv
        y = jnp.dot(agg, wl_ref[...], preferred_element_type=_f32) \
            + xw + bl_ref[...]
        (y0_ref if b == 0 else y1_ref)[...] = y
        _stats_update(st_ref, b, y, i)


def _dense_l1_body(*refs):
    hs = refs[0:2 * NCH]
    aggs = refs[2 * NCH:4 * NCH]
    c0_ref, c1_ref, wl_ref, wr_ref, bl_ref = refs[4 * NCH:5 + 4 * NCH]
    y0_ref, y1_ref, st_ref = refs[5 + 4 * NCH:]
    i = pl.program_id(0)
    for b in range(2):
        hc = hs[b * NCH:(b + 1) * NCH]
        ac = aggs[b * NCH:(b + 1) * NCH]
        cref = c0_ref if b == 0 else c1_ref
        xcat = jnp.concatenate([h[...] for h in hc], axis=1)
        inv = 1.0 / jnp.maximum(cref[0], 1.0)
        agg = jnp.concatenate([a[...] for a in ac], axis=1) * inv
        y = (jnp.dot(agg, wl_ref[...], preferred_element_type=_f32)
             + jnp.dot(xcat, wr_ref[...], preferred_element_type=_f32)
             + bl_ref[...])
        (y0_ref if b == 0 else y1_ref)[...] = y
        _stats_update(st_ref, b, y, i)


def _bn_body(*refs):
    y0_ref, y1_ref, st_ref, g_ref, bb_ref = refs[:5]
    h_refs = refs[5:]
    for b in range(2):
        st = st_ref[b]
        m = st[0:1, :] * (1.0 / N)
        var = st[1:2, :] * (1.0 / N) - m * m
        s = g_ref[...] * lax.rsqrt(var + 1e-5)
        t = bb_ref[...] - m * s
        y = (y0_ref if b == 0 else y1_ref)[...]
        h = jnp.maximum(y * s + t, 0.0)
        for c in range(NCH):
            h_refs[b * NCH + c][...] = h[:, c * CW:(c + 1) * CW]


def _final_body(*refs):
    aggs = refs[0:2 * NCH]
    c0_ref, c1_ref = refs[2 * NCH:2 + 2 * NCH]
    hs = refs[2 + 2 * NCH:2 + 4 * NCH]
    batch_ref, wl_ref, wr_ref, bl_ref = refs[2 + 4 * NCH:6 + 4 * NCH]
    out_ref, pool_a, pool_h, gcnt = refs[6 + 4 * NCH:]
    i = pl.program_id(0)
    inv0 = 1.0 / jnp.maximum(c0_ref[0], 1.0)
    inv1 = 1.0 / jnp.maximum(c1_ref[0], 1.0)
    agg = (jnp.concatenate([a[...] for a in aggs[:NCH]], axis=1) * inv0
           + jnp.concatenate([a[...] for a in aggs[NCH:]], axis=1) * inv1
           ) * 0.5
    h = (jnp.concatenate([x[...] for x in hs[:NCH]], axis=1)
         + jnp.concatenate([x[...] for x in hs[NCH:]], axis=1)) * 0.5
    bt = batch_ref[0]
    onehot = (bt == lax.broadcasted_iota(jnp.int32, (R, G), 1)).astype(_f32)
    dn = (((0,), (0,)), ((), ()))
    pa = lax.dot_general(onehot, agg, dn, preferred_element_type=_f32)
    ph = lax.dot_general(onehot, h, dn, preferred_element_type=_f32)
    gc = lax.dot_general(onehot, jnp.ones((R, 8), _f32), dn,
                         preferred_element_type=_f32)

    @pl.when(i == 0)
    def _():
        pool_a[...] = pa
        pool_h[...] = ph
        gcnt[...] = gc

    @pl.when(i != 0)
    def _():
        pool_a[...] += pa
        pool_h[...] += ph
        gcnt[...] += gc

    @pl.when(i == NB - 1)
    def _():
        gcv = jnp.maximum(gcnt[...][:, 0:1], 1.0)
        out_ref[...] = (
            jnp.dot(pool_a[...] / gcv, wl_ref[...],
                    preferred_element_type=_f32)
            + jnp.dot(pool_h[...] / gcv, wr_ref[...],
                      preferred_element_type=_f32)
            + bl_ref[...])


def _row_spec(w):
    return pl.BlockSpec((R, w), lambda i: (i, 0))


def _full_spec(r, c):
    return pl.BlockSpec((r, c), lambda i: (0, 0))


_st_spec = pl.BlockSpec((2, 8, D_H), lambda i: (0, 0, 0))
_cnt_spec = pl.BlockSpec((1, R, 1), lambda i: (i, 0, 0))

_dense_l0 = pl.pallas_call(
    _dense_l0_body,
    grid=(NB,),
    in_specs=[_row_spec(D_IN)] + [_row_spec(CW)] * (2 * NCX) + [
        _cnt_spec, _cnt_spec,
        _full_spec(D_IN, D_H), _full_spec(D_IN, D_H), _full_spec(1, D_H),
    ],
    out_specs=[_row_spec(D_H), _row_spec(D_H), _st_spec],
    out_shape=[
        jax.ShapeDtypeStruct((N, D_H), _f32),
        jax.ShapeDtypeStruct((N, D_H), _f32),
        jax.ShapeDtypeStruct((2, 8, D_H), _f32),
    ],
)

_dense_l1 = pl.pallas_call(
    _dense_l1_body,
    grid=(NB,),
    in_specs=[_row_spec(CW)] * (4 * NCH) + [
        _cnt_spec, _cnt_spec,
        _full_spec(D_H, D_H), _full_spec(D_H, D_H), _full_spec(1, D_H),
    ],
    out_specs=[_row_spec(D_H), _row_spec(D_H), _st_spec],
    out_shape=[
        jax.ShapeDtypeStruct((N, D_H), _f32),
        jax.ShapeDtypeStruct((N, D_H), _f32),
        jax.ShapeDtypeStruct((2, 8, D_H), _f32),
    ],
)

_bn_relu = pl.pallas_call(
    _bn_body,
    grid=(NB,),
    in_specs=[
        _row_spec(D_H), _row_spec(D_H), _st_spec,
        _full_spec(1, D_H), _full_spec(1, D_H),
    ],
    out_specs=[_row_spec(CW)] * (2 * NCH),
    out_shape=[jax.ShapeDtypeStruct((N, CW), _f32)] * (2 * NCH),
)

_final = pl.pallas_call(
    _final_body,
    grid=(NB,),
    in_specs=[_row_spec(CW)] * (2 * NCH) + [
        _cnt_spec, _cnt_spec,
    ] + [_row_spec(CW)] * (2 * NCH) + [
        pl.BlockSpec((1, R, 1), lambda i: (i, 0, 0)),
        _full_spec(D_H, D_H), _full_spec(D_H, D_H), _full_spec(1, D_H),
    ],
    out_specs=pl.BlockSpec((G, D_H), lambda i: (0, 0)),
    out_shape=jax.ShapeDtypeStruct((G, D_H), _f32),
    scratch_shapes=[
        pltpu.VMEM((G, D_H), _f32),
        pltpu.VMEM((G, D_H), _f32),
        pltpu.VMEM((G, 8), _f32),
    ],
)


def kernel(x, edge_index_sc, edge_index_fc, batch,
           Wl0, bl0, Wr0, g0, b0,
           Wl1, bl1, Wr1, g1, b1,
           Wl2, bl2, Wr2):
    xc = [x[:, c * CW:(c + 1) * CW] for c in range(NCX)]
    z = jnp.zeros((N, CW), _f32)

    def edges(ei):
        return (ei[0].reshape(NT, NCHUNK, K), ei[1].reshape(NT, NCHUNK, K))

    src0, dst0 = edges(edge_index_sc)
    src1, dst1 = edges(edge_index_fc)

    ones_rows = jnp.ones((K, CW), _f32)
    cnt0_t, cnt1_t = _sc_cnt(dst0, dst1, z, ones_rows)
    cnt0 = cnt0_t[:, 0].reshape(NB, R, 1)
    cnt1 = cnt1_t[:, 0].reshape(NB, R, 1)

    a0_0 = _sc_l0(*xc, src0, dst0, z)
    a0_1 = _sc_l0(*xc, src1, dst1, z)

    y00, y01, st0 = _dense_l0(x, *a0_0, *a0_1, cnt0, cnt1,
                              Wl0, Wr0, bl0.reshape(1, D_H))
    h0 = _bn_relu(y00, y01, st0, g0.reshape(1, D_H), b0.reshape(1, D_H))

    agg1_0 = _sc_agg(*h0[:NCH], src0, dst0, z)
    agg1_1 = _sc_agg(*h0[NCH:], src1, dst1, z)

    y10, y11, st1 = _dense_l1(*h0, *agg1_0, *agg1_1, cnt0, cnt1,
                              Wl1, Wr1, bl1.reshape(1, D_H))
    h1 = _bn_relu(y10, y11, st1, g1.reshape(1, D_H), b1.reshape(1, D_H))

    agg2_0 = _sc_agg(*h1[:NCH], src0, dst0, z)
    agg2_1 = _sc_agg(*h1[NCH:], src1, dst1, z)

    out = _final(*agg2_0, *agg2_1, cnt0, cnt1, *h1,
                 batch.reshape(NB, R, 1), Wl2, Wr2, bl2.reshape(1, D_H))
    return out


# speedup vs baseline: 3.3513x; 1.2107x over previous
import jax
import jax.numpy as jnp
from jax import lax
from jax.experimental import pallas as pl
from jax.experimental.pallas import tpu as pltpu
from jax.experimental.pallas import tpu_sc as plsc

N = 10000
E = 160000
D_IN = 256
D_H = 512
G = 64

CW = 128
NCX = D_IN // CW
NCH = D_H // CW
NT = 16
K = 50
NCHUNK = 200
NGRP = 5
GCH = NCHUNK // NGRP
RPT = 624
RTAIL = N - NT * RPT
R = 1000
NB = N // R

_mesh = plsc.VectorSubcoreMesh(core_axis_name="c", subcore_axis_name="s")
_f32 = jnp.float32


def _part_copy(src, dst, sid, cond=None):
    sl = pl.ds(sid * RPT, RPT)
    tl = pl.ds(NT * RPT, RTAIL)
    is_last = sid == NT - 1
    if cond is None:
        pltpu.sync_copy(src.at[sl], dst.at[sl])

        @pl.when(is_last)
        def _():
            pltpu.sync_copy(src.at[tl], dst.at[tl])
    else:
        @pl.when(cond)
        def _():
            pltpu.sync_copy(src.at[sl], dst.at[sl])

        @pl.when(cond & is_last)
        def _():
            pltpu.sync_copy(src.at[tl], dst.at[tl])


def _sc_cnt_body(dst0, dst1, z, ones_h, cnt0, cnt1,
                 dst_v, ones_v, acc):
    cid = lax.axis_index("c")
    sid = lax.axis_index("s")
    _part_copy(z, acc, sid)
    pltpu.sync_copy(ones_h, ones_v)
    plsc.subcore_barrier()

    def loop(j, carry):
        pltpu.sync_copy(ones_v, acc.at[dst_v.at[j]], add=True)
        return carry

    for g in range(NGRP):
        @pl.when(cid == 0)
        def _():
            pltpu.sync_copy(dst0.at[sid, pl.ds(g * GCH, GCH)], dst_v)

        @pl.when(cid == 1)
        def _():
            pltpu.sync_copy(dst1.at[sid, pl.ds(g * GCH, GCH)], dst_v)

        lax.fori_loop(0, GCH, loop, 0)

    plsc.subcore_barrier()
    _part_copy(acc, cnt0, sid, cond=cid == 0)
    _part_copy(acc, cnt1, sid, cond=cid == 1)


_sc_cnt = pl.kernel(
    _sc_cnt_body,
    out_type=(jax.ShapeDtypeStruct((N, CW), _f32),
              jax.ShapeDtypeStruct((N, CW), _f32)),
    mesh=_mesh,
    scratch_types=[
        pltpu.VMEM((GCH, K), jnp.int32),
        pltpu.VMEM((K, CW), _f32),
        pltpu.VMEM_SHARED((N, CW), _f32),
    ],
)


def _make_sc_seg(nchunks):
    npass = nchunks // 2

    def body(*refs):
        pos = 0
        hs = refs[pos:pos + nchunks]; pos += nchunks
        srcr, dstr, z = refs[pos:pos + 3]; pos += 3
        outs = refs[pos:pos + nchunks]; pos += nchunks
        src_v, dst_v, rows_a, rows_b, sem_a, sem_b, acc = refs[pos:pos + 7]

        cid = lax.axis_index("c")
        sid = lax.axis_index("s")

        def pipe_loop(tbl):
            def start(j, buf, sem):
                pltpu.async_copy(tbl.at[src_v.at[j]], buf, sem)

            def wait(j, buf, sem):
                pltpu.make_async_copy(tbl.at[src_v.at[j]], buf, sem).wait()

            def scat(j, buf):
                pltpu.sync_copy(buf, acc.at[dst_v.at[j]], add=True)

            start(0, rows_a, sem_a)

            def body2(k, carry):
                ja = 2 * k
                jb = 2 * k + 1
                jn = jnp.minimum(2 * k + 2, GCH - 1)
                wait(ja, rows_a, sem_a)
                start(jb, rows_b, sem_b)
                scat(ja, rows_a)
                wait(jb, rows_b, sem_b)
                start(jn, rows_a, sem_a)
                scat(jb, rows_b)
                return carry

            lax.fori_loop(0, GCH // 2, body2, 0)
            wait(GCH - 1, rows_a, sem_a)

        for p in range(npass):
            h_sc0 = hs[p]
            h_sc1 = hs[npass + p]

            _part_copy(z, acc, sid)
            plsc.subcore_barrier()

            for g in range(NGRP):
                pltpu.sync_copy(srcr.at[sid, pl.ds(g * GCH, GCH)], src_v)
                pltpu.sync_copy(dstr.at[sid, pl.ds(g * GCH, GCH)], dst_v)

                @pl.when(cid == 0)
                def _():
                    pipe_loop(h_sc0)

                @pl.when(cid == 1)
                def _():
                    pipe_loop(h_sc1)

            plsc.subcore_barrier()
            _part_copy(acc, outs[p], sid, cond=cid == 0)
            _part_copy(acc, outs[npass + p], sid, cond=cid == 1)
            plsc.subcore_barrier()

    out_type = [jax.ShapeDtypeStruct((N, CW), _f32) for _ in range(nchunks)]
    scratch = [
        pltpu.VMEM((GCH, K), jnp.int32),
        pltpu.VMEM((GCH, K), jnp.int32),
        pltpu.VMEM((K, CW), _f32),
        pltpu.VMEM((K, CW), _f32),
        pltpu.SemaphoreType.DMA,
        pltpu.SemaphoreType.DMA,
        pltpu.VMEM_SHARED((N, CW), _f32),
    ]
    return pl.kernel(body, out_type=tuple(out_type), mesh=_mesh,
                     scratch_types=scratch)


_sc_l0 = _make_sc_seg(NCX)
_sc_agg = _make_sc_seg(NCH)


def _stats_update(st_ref, b, y, i):
    st = jnp.concatenate(
        [jnp.sum(y, axis=0, keepdims=True),
         jnp.sum(y * y, axis=0, keepdims=True),
         jnp.zeros((6, D_H), _f32)], axis=0)

    @pl.when(i == 0)
    def _():
        st_ref[b] = st

    @pl.when(i != 0)
    def _():
        st_ref[b] += st


def _dense_l0_body(*refs):
    x_ref = refs[0]
    aggs = refs[1:1 + 2 * NCX]
    c0_ref, c1_ref, wl_ref, wr_ref, bl_ref = refs[1 + 2 * NCX:6 + 2 * NCX]
    y0_ref, y1_ref, st_ref = refs[6 + 2 * NCX:]
    i = pl.program_id(0)
    xw = jnp.dot(x_ref[...], wr_ref[...], preferred_element_type=_f32)
    for b in range(2):
        ac = aggs[b * NCX:(b + 1) * NCX]
        cref = c0_ref if b == 0 else c1_ref
        inv = 1.0 / jnp.maximum(cref[0], 1.0)
        agg = jnp.concatenate([a[...] for a in ac], axis=1) * inv
        y = jnp.dot(agg, wl_ref[...], preferred_element_type=_f32) \
            + xw + bl_ref[...]
        (y0_ref if b == 0 else y1_ref)[...] = y
        _stats_update(st_ref, b, y, i)


def _dense_l1_body(*refs):
    hs = refs[0:2 * NCH]
    aggs = refs[2 * NCH:4 * NCH]
    c0_ref, c1_ref, wl_ref, wr_ref, bl_ref = refs[4 * NCH:5 + 4 * NCH]
    y0_ref, y1_ref, st_ref = refs[5 + 4 * NCH:]
    i = pl.program_id(0)
    for b in range(2):
        hc = hs[b * NCH:(b + 1) * NCH]
        ac = aggs[b * NCH:(b + 1) * NCH]
        cref = c0_ref if b == 0 else c1_ref
        xcat = jnp.concatenate([h[...] for h in hc], axis=1)
        inv = 1.0 / jnp.maximum(cref[0], 1.0)
        agg = jnp.concatenate([a[...] for a in ac], axis=1) * inv
        y = (jnp.dot(agg, wl_ref[...], preferred_element_type=_f32)
             + jnp.dot(xcat, wr_ref[...], preferred_element_type=_f32)
             + bl_ref[...])
        (y0_ref if b == 0 else y1_ref)[...] = y
        _stats_update(st_ref, b, y, i)


def _bn_body(*refs):
    y0_ref, y1_ref, st_ref, g_ref, bb_ref = refs[:5]
    h_refs = refs[5:]
    for b in range(2):
        st = st_ref[b]
        m = st[0:1, :] * (1.0 / N)
        var = st[1:2, :] * (1.0 / N) - m * m
        s = g_ref[...] * lax.rsqrt(var + 1e-5)
        t = bb_ref[...] - m * s
        y = (y0_ref if b == 0 else y1_ref)[...]
        h = jnp.maximum(y * s + t, 0.0)
        for c in range(NCH):
            h_refs[b * NCH + c][...] = h[:, c * CW:(c + 1) * CW]


def _final_body(*refs):
    aggs = refs[0:2 * NCH]
    c0_ref, c1_ref = refs[2 * NCH:2 + 2 * NCH]
    hs = refs[2 + 2 * NCH:2 + 4 * NCH]
    batch_ref, wl_ref, wr_ref, bl_ref = refs[2 + 4 * NCH:6 + 4 * NCH]
    out_ref, pool_a, pool_h, gcnt = refs[6 + 4 * NCH:]
    i = pl.program_id(0)
    inv0 = 1.0 / jnp.maximum(c0_ref[0], 1.0)
    inv1 = 1.0 / jnp.maximum(c1_ref[0], 1.0)
    agg = (jnp.concatenate([a[...] for a in aggs[:NCH]], axis=1) * inv0
           + jnp.concatenate([a[...] for a in aggs[NCH:]], axis=1) * inv1
           ) * 0.5
    h = (jnp.concatenate([x[...] for x in hs[:NCH]], axis=1)
         + jnp.concatenate([x[...] for x in hs[NCH:]], axis=1)) * 0.5
    bt = batch_ref[0]
    onehot = (bt == lax.broadcasted_iota(jnp.int32, (R, G), 1)).astype(_f32)
    dn = (((0,), (0,)), ((), ()))
    pa = lax.dot_general(onehot, agg, dn, preferred_element_type=_f32)
    ph = lax.dot_general(onehot, h, dn, preferred_element_type=_f32)
    gc = lax.dot_general(onehot, jnp.ones((R, 8), _f32), dn,
                         preferred_element_type=_f32)

    @pl.when(i == 0)
    def _():
        pool_a[...] = pa
        pool_h[...] = ph
        gcnt[...] = gc

    @pl.when(i != 0)
    def _():
        pool_a[...] += pa
        pool_h[...] += ph
        gcnt[...] += gc

    @pl.when(i == NB - 1)
    def _():
        gcv = jnp.maximum(gcnt[...][:, 0:1], 1.0)
        out_ref[...] = (
            jnp.dot(pool_a[...] / gcv, wl_ref[...],
                    preferred_element_type=_f32)
            + jnp.dot(pool_h[...] / gcv, wr_ref[...],
                      preferred_element_type=_f32)
            + bl_ref[...])


def _row_spec(w):
    return pl.BlockSpec((R, w), lambda i: (i, 0))


def _full_spec(r, c):
    return pl.BlockSpec((r, c), lambda i: (0, 0))


_st_spec = pl.BlockSpec((2, 8, D_H), lambda i: (0, 0, 0))
_cnt_spec = pl.BlockSpec((1, R, 1), lambda i: (i, 0, 0))

_dense_l0 = pl.pallas_call(
    _dense_l0_body,
    grid=(NB,),
    in_specs=[_row_spec(D_IN)] + [_row_spec(CW)] * (2 * NCX) + [
        _cnt_spec, _cnt_spec,
        _full_spec(D_IN, D_H), _full_spec(D_IN, D_H), _full_spec(1, D_H),
    ],
    out_specs=[_row_spec(D_H), _row_spec(D_H), _st_spec],
    out_shape=[
        jax.ShapeDtypeStruct((N, D_H), _f32),
        jax.ShapeDtypeStruct((N, D_H), _f32),
        jax.ShapeDtypeStruct((2, 8, D_H), _f32),
    ],
)

_dense_l1 = pl.pallas_call(
    _dense_l1_body,
    grid=(NB,),
    in_specs=[_row_spec(CW)] * (4 * NCH) + [
        _cnt_spec, _cnt_spec,
        _full_spec(D_H, D_H), _full_spec(D_H, D_H), _full_spec(1, D_H),
    ],
    out_specs=[_row_spec(D_H), _row_spec(D_H), _st_spec],
    out_shape=[
        jax.ShapeDtypeStruct((N, D_H), _f32),
        jax.ShapeDtypeStruct((N, D_H), _f32),
        jax.ShapeDtypeStruct((2, 8, D_H), _f32),
    ],
)

_bn_relu = pl.pallas_call(
    _bn_body,
    grid=(NB,),
    in_specs=[
        _row_spec(D_H), _row_spec(D_H), _st_spec,
        _full_spec(1, D_H), _full_spec(1, D_H),
    ],
    out_specs=[_row_spec(CW)] * (2 * NCH),
    out_shape=[jax.ShapeDtypeStruct((N, CW), _f32)] * (2 * NCH),
)

_final = pl.pallas_call(
    _final_body,
    grid=(NB,),
    in_specs=[_row_spec(CW)] * (2 * NCH) + [
        _cnt_spec, _cnt_spec,
    ] + [_row_spec(CW)] * (2 * NCH) + [
        pl.BlockSpec((1, R, 1), lambda i: (i, 0, 0)),
        _full_spec(D_H, D_H), _full_spec(D_H, D_H), _full_spec(1, D_H),
    ],
    out_specs=pl.BlockSpec((G, D_H), lambda i: (0, 0)),
    out_shape=jax.ShapeDtypeStruct((G, D_H), _f32),
    scratch_shapes=[
        pltpu.VMEM((G, D_H), _f32),
        pltpu.VMEM((G, D_H), _f32),
        pltpu.VMEM((G, 8), _f32),
    ],
)


def kernel(x, edge_index_sc, edge_index_fc, batch,
           Wl0, bl0, Wr0, g0, b0,
           Wl1, bl1, Wr1, g1, b1,
           Wl2, bl2, Wr2):
    xc = [x[:, c * CW:(c + 1) * CW] for c in range(NCX)]
    z = jnp.zeros((N, CW), _f32)

    def edges(ei):
        return (ei[0].reshape(NT, NCHUNK, K), ei[1].reshape(NT, NCHUNK, K))

    src0, dst0 = edges(edge_index_sc)
    src1, dst1 = edges(edge_index_fc)

    ones_rows = jnp.ones((K, CW), _f32)
    cnt0_t, cnt1_t = _sc_cnt(dst0, dst1, z, ones_rows)
    cnt0 = cnt0_t[:, 0].reshape(NB, R, 1)
    cnt1 = cnt1_t[:, 0].reshape(NB, R, 1)

    a0_0 = _sc_l0(*xc, src0, dst0, z)
    a0_1 = _sc_l0(*xc, src1, dst1, z)

    y00, y01, st0 = _dense_l0(x, *a0_0, *a0_1, cnt0, cnt1,
                              Wl0, Wr0, bl0.reshape(1, D_H))
    h0 = _bn_relu(y00, y01, st0, g0.reshape(1, D_H), b0.reshape(1, D_H))

    agg1_0 = _sc_agg(*h0[:NCH], src0, dst0, z)
    agg1_1 = _sc_agg(*h0[NCH:], src1, dst1, z)

    y10, y11, st1 = _dense_l1(*h0, *agg1_0, *agg1_1, cnt0, cnt1,
                              Wl1, Wr1, bl1.reshape(1, D_H))
    h1 = _bn_relu(y10, y11, st1, g1.reshape(1, D_H), b1.reshape(1, D_H))

    agg2_0 = _sc_agg(*h1[:NCH], src0, dst0, z)
    agg2_1 = _sc_agg(*h1[NCH:], src1, dst1, z)

    out = _final(*agg2_0, *agg2_1, cnt0, cnt1, *h1,
                 batch.reshape(NB, R, 1), Wl2, Wr2, bl2.reshape(1, D_H))
    return out


# speedup vs baseline: 3.7959x; 1.1326x over previous
import jax
import jax.numpy as jnp
from jax import lax
from jax.experimental import pallas as pl
from jax.experimental.pallas import tpu as pltpu
from jax.experimental.pallas import tpu_sc as plsc

N = 10000
E = 160000
D_IN = 256
D_H = 512
G = 64

CW = 128
NCX = D_IN // CW
NCH = D_H // CW
NT = 16
K = 50
NCHUNK = 200
NGRP = 5
GCH = NCHUNK // NGRP
RPT = 624
RTAIL = N - NT * RPT
R = 1000
NB = N // R

_mesh = plsc.VectorSubcoreMesh(core_axis_name="c", subcore_axis_name="s")
_f32 = jnp.float32


def _part_copy(src, dst, sid, cond=None):
    sl = pl.ds(sid * RPT, RPT)
    tl = pl.ds(NT * RPT, RTAIL)
    is_last = sid == NT - 1
    if cond is None:
        pltpu.sync_copy(src.at[sl], dst.at[sl])

        @pl.when(is_last)
        def _():
            pltpu.sync_copy(src.at[tl], dst.at[tl])
    else:
        @pl.when(cond)
        def _():
            pltpu.sync_copy(src.at[sl], dst.at[sl])

        @pl.when(cond & is_last)
        def _():
            pltpu.sync_copy(src.at[tl], dst.at[tl])


def _sc_cnt_body(dst0, dst1, z, ones_h, cnt0, cnt1,
                 dst_v, ones_v, sem_a, sem_b, acc):
    cid = lax.axis_index("c")
    sid = lax.axis_index("s")
    _part_copy(z, acc, sid)
    pltpu.sync_copy(ones_h, ones_v)
    plsc.subcore_barrier()

    def s_start(j, sem):
        pltpu.async_copy(ones_v, acc.at[dst_v.at[j]], sem, add=True)

    def s_wait(j, sem):
        pltpu.make_async_copy(ones_v, acc.at[dst_v.at[j]], sem).wait()

    def pipe(k, carry):
        ja = 2 * k
        jb = 2 * k + 1
        s_wait(ja, sem_a)
        s_start(ja + 2, sem_a)
        s_wait(jb, sem_b)
        s_start(jb + 2, sem_b)
        return carry

    for g in range(NGRP):
        @pl.when(cid == 0)
        def _():
            pltpu.sync_copy(dst0.at[sid, pl.ds(g * GCH, GCH)], dst_v)

        @pl.when(cid == 1)
        def _():
            pltpu.sync_copy(dst1.at[sid, pl.ds(g * GCH, GCH)], dst_v)

        s_start(0, sem_a)
        s_start(1, sem_b)
        lax.fori_loop(0, GCH // 2 - 1, pipe, 0)
        s_wait(GCH - 2, sem_a)
        s_wait(GCH - 1, sem_b)

    plsc.subcore_barrier()
    _part_copy(acc, cnt0, sid, cond=cid == 0)
    _part_copy(acc, cnt1, sid, cond=cid == 1)


_sc_cnt = pl.kernel(
    _sc_cnt_body,
    out_type=(jax.ShapeDtypeStruct((N, CW), _f32),
              jax.ShapeDtypeStruct((N, CW), _f32)),
    mesh=_mesh,
    scratch_types=[
        pltpu.VMEM((GCH, K), jnp.int32),
        pltpu.VMEM((K, CW), _f32),
        pltpu.SemaphoreType.DMA,
        pltpu.SemaphoreType.DMA,
        pltpu.VMEM_SHARED((N, CW), _f32),
    ],
)


def _make_sc_seg(nchunks):
    npass = nchunks // 2

    def body(*refs):
        pos = 0
        hs = refs[pos:pos + nchunks]; pos += nchunks
        srcr, dstr, z = refs[pos:pos + 3]; pos += 3
        outs = refs[pos:pos + nchunks]; pos += nchunks
        (src_v, dst_v, rows_a, rows_b,
         sem_a, sem_b, sem_sa, sem_sb, acc) = refs[pos:pos + 9]

        cid = lax.axis_index("c")
        sid = lax.axis_index("s")

        def pipe_loop(tbl):
            def g_start(j, buf, sem):
                pltpu.async_copy(tbl.at[src_v.at[j]], buf, sem)

            def g_wait(j, buf, sem):
                pltpu.make_async_copy(tbl.at[src_v.at[j]], buf, sem).wait()

            def s_start(j, buf, sem):
                pltpu.async_copy(buf, acc.at[dst_v.at[j]], sem, add=True)

            def s_wait(j, buf, sem):
                pltpu.make_async_copy(buf, acc.at[dst_v.at[j]], sem).wait()

            g_start(0, rows_a, sem_a)
            g_start(1, rows_b, sem_b)

            def body2(k, carry):
                ja = 2 * k
                jb = 2 * k + 1
                jn_a = jnp.minimum(ja + 2, GCH - 1)
                jn_b = jnp.minimum(jb + 2, GCH - 1)
                g_wait(ja, rows_a, sem_a)
                s_start(ja, rows_a, sem_sa)
                g_wait(jb, rows_b, sem_b)
                s_start(jb, rows_b, sem_sb)
                s_wait(ja, rows_a, sem_sa)
                g_start(jn_a, rows_a, sem_a)
                s_wait(jb, rows_b, sem_sb)
                g_start(jn_b, rows_b, sem_b)
                return carry

            lax.fori_loop(0, GCH // 2, body2, 0)
            g_wait(GCH - 1, rows_a, sem_a)
            g_wait(GCH - 1, rows_b, sem_b)

        for p in range(npass):
            h_sc0 = hs[p]
            h_sc1 = hs[npass + p]

            _part_copy(z, acc, sid)
            plsc.subcore_barrier()

            for g in range(NGRP):
                pltpu.sync_copy(srcr.at[sid, pl.ds(g * GCH, GCH)], src_v)
                pltpu.sync_copy(dstr.at[sid, pl.ds(g * GCH, GCH)], dst_v)

                @pl.when(cid == 0)
                def _():
                    pipe_loop(h_sc0)

                @pl.when(cid == 1)
                def _():
                    pipe_loop(h_sc1)

            plsc.subcore_barrier()
            _part_copy(acc, outs[p], sid, cond=cid == 0)
            _part_copy(acc, outs[npass + p], sid, cond=cid == 1)
            plsc.subcore_barrier()

    out_type = [jax.ShapeDtypeStruct((N, CW), _f32) for _ in range(nchunks)]
    scratch = [
        pltpu.VMEM((GCH, K), jnp.int32),
        pltpu.VMEM((GCH, K), jnp.int32),
        pltpu.VMEM((K, CW), _f32),
        pltpu.VMEM((K, CW), _f32),
        pltpu.SemaphoreType.DMA,
        pltpu.SemaphoreType.DMA,
        pltpu.SemaphoreType.DMA,
        pltpu.SemaphoreType.DMA,
        pltpu.VMEM_SHARED((N, CW), _f32),
    ]
    return pl.kernel(body, out_type=tuple(out_type), mesh=_mesh,
                     scratch_types=scratch)


_sc_l0 = _make_sc_seg(NCX)
_sc_agg = _make_sc_seg(NCH)


def _stats_update(st_ref, b, y, i):
    st = jnp.concatenate(
        [jnp.sum(y, axis=0, keepdims=True),
         jnp.sum(y * y, axis=0, keepdims=True),
         jnp.zeros((6, D_H), _f32)], axis=0)

    @pl.when(i == 0)
    def _():
        st_ref[b] = st

    @pl.when(i != 0)
    def _():
        st_ref[b] += st


def _dense_l0_body(*refs):
    x_ref = refs[0]
    aggs = refs[1:1 + 2 * NCX]
    c0_ref, c1_ref, wl_ref, wr_ref, bl_ref = refs[1 + 2 * NCX:6 + 2 * NCX]
    y0_ref, y1_ref, st_ref = refs[6 + 2 * NCX:]
    i = pl.program_id(0)
    xw = jnp.dot(x_ref[...], wr_ref[...], preferred_element_type=_f32)
    for b in range(2):
        ac = aggs[b * NCX:(b + 1) * NCX]
        cref = c0_ref if b == 0 else c1_ref
        inv = 1.0 / jnp.maximum(cref[0], 1.0)
        agg = jnp.concatenate([a[...] for a in ac], axis=1) * inv
        y = jnp.dot(agg, wl_ref[...], preferred_element_type=_f32) \
            + xw + bl_ref[...]
        (y0_ref if b == 0 else y1_ref)[...] = y
        _stats_update(st_ref, b, y, i)


def _dense_l1_body(*refs):
    hs = refs[0:2 * NCH]
    aggs = refs[2 * NCH:4 * NCH]
    c0_ref, c1_ref, wl_ref, wr_ref, bl_ref = refs[4 * NCH:5 + 4 * NCH]
    y0_ref, y1_ref, st_ref = refs[5 + 4 * NCH:]
    i = pl.program_id(0)
    for b in range(2):
        hc = hs[b * NCH:(b + 1) * NCH]
        ac = aggs[b * NCH:(b + 1) * NCH]
        cref = c0_ref if b == 0 else c1_ref
        xcat = jnp.concatenate([h[...] for h in hc], axis=1)
        inv = 1.0 / jnp.maximum(cref[0], 1.0)
        agg = jnp.concatenate([a[...] for a in ac], axis=1) * inv
        y = (jnp.dot(agg, wl_ref[...], preferred_element_type=_f32)
             + jnp.dot(xcat, wr_ref[...], preferred_element_type=_f32)
             + bl_ref[...])
        (y0_ref if b == 0 else y1_ref)[...] = y
        _stats_update(st_ref, b, y, i)


def _bn_body(*refs):
    y0_ref, y1_ref, st_ref, g_ref, bb_ref = refs[:5]
    h_refs = refs[5:]
    for b in range(2):
        st = st_ref[b]
        m = st[0:1, :] * (1.0 / N)
        var = st[1:2, :] * (1.0 / N) - m * m
        s = g_ref[...] * lax.rsqrt(var + 1e-5)
        t = bb_ref[...] - m * s
        y = (y0_ref if b == 0 else y1_ref)[...]
        h = jnp.maximum(y * s + t, 0.0)
        for c in range(NCH):
            h_refs[b * NCH + c][...] = h[:, c * CW:(c + 1) * CW]


def _final_body(*refs):
    aggs = refs[0:2 * NCH]
    c0_ref, c1_ref = refs[2 * NCH:2 + 2 * NCH]
    hs = refs[2 + 2 * NCH:2 + 4 * NCH]
    batch_ref, wl_ref, wr_ref, bl_ref = refs[2 + 4 * NCH:6 + 4 * NCH]
    out_ref, pool_a, pool_h, gcnt = refs[6 + 4 * NCH:]
    i = pl.program_id(0)
    inv0 = 1.0 / jnp.maximum(c0_ref[0], 1.0)
    inv1 = 1.0 / jnp.maximum(c1_ref[0], 1.0)
    agg = (jnp.concatenate([a[...] for a in aggs[:NCH]], axis=1) * inv0
           + jnp.concatenate([a[...] for a in aggs[NCH:]], axis=1) * inv1
           ) * 0.5
    h = (jnp.concatenate([x[...] for x in hs[:NCH]], axis=1)
         + jnp.concatenate([x[...] for x in hs[NCH:]], axis=1)) * 0.5
    bt = batch_ref[0]
    onehot = (bt == lax.broadcasted_iota(jnp.int32, (R, G), 1)).astype(_f32)
    dn = (((0,), (0,)), ((), ()))
    pa = lax.dot_general(onehot, agg, dn, preferred_element_type=_f32)
    ph = lax.dot_general(onehot, h, dn, preferred_element_type=_f32)
    gc = lax.dot_general(onehot, jnp.ones((R, 8), _f32), dn,
                         preferred_element_type=_f32)

    @pl.when(i == 0)
    def _():
        pool_a[...] = pa
        pool_h[...] = ph
        gcnt[...] = gc

    @pl.when(i != 0)
    def _():
        pool_a[...] += pa
        pool_h[...] += ph
        gcnt[...] += gc

    @pl.when(i == NB - 1)
    def _():
        gcv = jnp.maximum(gcnt[...][:, 0:1], 1.0)
        out_ref[...] = (
            jnp.dot(pool_a[...] / gcv, wl_ref[...],
                    preferred_element_type=_f32)
            + jnp.dot(pool_h[...] / gcv, wr_ref[...],
                      preferred_element_type=_f32)
            + bl_ref[...])


def _row_spec(w):
    return pl.BlockSpec((R, w), lambda i: (i, 0))


def _full_spec(r, c):
    return pl.BlockSpec((r, c), lambda i: (0, 0))


_st_spec = pl.BlockSpec((2, 8, D_H), lambda i: (0, 0, 0))
_cnt_spec = pl.BlockSpec((1, R, 1), lambda i: (i, 0, 0))

_dense_l0 = pl.pallas_call(
    _dense_l0_body,
    grid=(NB,),
    in_specs=[_row_spec(D_IN)] + [_row_spec(CW)] * (2 * NCX) + [
        _cnt_spec, _cnt_spec,
        _full_spec(D_IN, D_H), _full_spec(D_IN, D_H), _full_spec(1, D_H),
    ],
    out_specs=[_row_spec(D_H), _row_spec(D_H), _st_spec],
    out_shape=[
        jax.ShapeDtypeStruct((N, D_H), _f32),
        jax.ShapeDtypeStruct((N, D_H), _f32),
        jax.ShapeDtypeStruct((2, 8, D_H), _f32),
    ],
)

_dense_l1 = pl.pallas_call(
    _dense_l1_body,
    grid=(NB,),
    in_specs=[_row_spec(CW)] * (4 * NCH) + [
        _cnt_spec, _cnt_spec,
        _full_spec(D_H, D_H), _full_spec(D_H, D_H), _full_spec(1, D_H),
    ],
    out_specs=[_row_spec(D_H), _row_spec(D_H), _st_spec],
    out_shape=[
        jax.ShapeDtypeStruct((N, D_H), _f32),
        jax.ShapeDtypeStruct((N, D_H), _f32),
        jax.ShapeDtypeStruct((2, 8, D_H), _f32),
    ],
)

_bn_relu = pl.pallas_call(
    _bn_body,
    grid=(NB,),
    in_specs=[
        _row_spec(D_H), _row_spec(D_H), _st_spec,
        _full_spec(1, D_H), _full_spec(1, D_H),
    ],
    out_specs=[_row_spec(CW)] * (2 * NCH),
    out_shape=[jax.ShapeDtypeStruct((N, CW), _f32)] * (2 * NCH),
)

_final = pl.pallas_call(
    _final_body,
    grid=(NB,),
    in_specs=[_row_spec(CW)] * (2 * NCH) + [
        _cnt_spec, _cnt_spec,
    ] + [_row_spec(CW)] * (2 * NCH) + [
        pl.BlockSpec((1, R, 1), lambda i: (i, 0, 0)),
        _full_spec(D_H, D_H), _full_spec(D_H, D_H), _full_spec(1, D_H),
    ],
    out_specs=pl.BlockSpec((G, D_H), lambda i: (0, 0)),
    out_shape=jax.ShapeDtypeStruct((G, D_H), _f32),
    scratch_shapes=[
        pltpu.VMEM((G, D_H), _f32),
        pltpu.VMEM((G, D_H), _f32),
        pltpu.VMEM((G, 8), _f32),
    ],
)


def kernel(x, edge_index_sc, edge_index_fc, batch,
           Wl0, bl0, Wr0, g0, b0,
           Wl1, bl1, Wr1, g1, b1,
           Wl2, bl2, Wr2):
    xc = [x[:, c * CW:(c + 1) * CW] for c in range(NCX)]
    z = jnp.zeros((N, CW), _f32)

    def edges(ei):
        return (ei[0].reshape(NT, NCHUNK, K), ei[1].reshape(NT, NCHUNK, K))

    src0, dst0 = edges(edge_index_sc)
    src1, dst1 = edges(edge_index_fc)

    ones_rows = jnp.ones((K, CW), _f32)
    cnt0_t, cnt1_t = _sc_cnt(dst0, dst1, z, ones_rows)
    cnt0 = cnt0_t[:, 0].reshape(NB, R, 1)
    cnt1 = cnt1_t[:, 0].reshape(NB, R, 1)

    a0_0 = _sc_l0(*xc, src0, dst0, z)
    a0_1 = _sc_l0(*xc, src1, dst1, z)

    y00, y01, st0 = _dense_l0(x, *a0_0, *a0_1, cnt0, cnt1,
                              Wl0, Wr0, bl0.reshape(1, D_H))
    h0 = _bn_relu(y00, y01, st0, g0.reshape(1, D_H), b0.reshape(1, D_H))

    agg1_0 = _sc_agg(*h0[:NCH], src0, dst0, z)
    agg1_1 = _sc_agg(*h0[NCH:], src1, dst1, z)

    y10, y11, st1 = _dense_l1(*h0, *agg1_0, *agg1_1, cnt0, cnt1,
                              Wl1, Wr1, bl1.reshape(1, D_H))
    h1 = _bn_relu(y10, y11, st1, g1.reshape(1, D_H), b1.reshape(1, D_H))

    agg2_0 = _sc_agg(*h1[:NCH], src0, dst0, z)
    agg2_1 = _sc_agg(*h1[NCH:], src1, dst1, z)

    out = _final(*agg2_0, *agg2_1, cnt0, cnt1, *h1,
                 batch.reshape(NB, R, 1), Wl2, Wr2, bl2.reshape(1, D_H))
    return out


# speedup vs baseline: 4.2094x; 1.1089x over previous
import jax
import jax.numpy as jnp
from jax import lax
from jax.experimental import pallas as pl
from jax.experimental.pallas import tpu as pltpu
from jax.experimental.pallas import tpu_sc as plsc

N = 10000
E = 160000
D_IN = 256
D_H = 512
G = 64

CW = 128
NCX = D_IN // CW
NCH = D_H // CW
NT = 16
K = 80
NCHUNK = 125
GCH = 40
NGRP = 3
GTAIL = NCHUNK - NGRP * GCH
RPT = 624
RTAIL = N - NT * RPT
R = 1000
NB = N // R

_mesh = plsc.VectorSubcoreMesh(core_axis_name="c", subcore_axis_name="s")
_f32 = jnp.float32


def _part_copy(src, dst, sid, cond=None):
    sl = pl.ds(sid * RPT, RPT)
    tl = pl.ds(NT * RPT, RTAIL)
    is_last = sid == NT - 1
    if cond is None:
        pltpu.sync_copy(src.at[sl], dst.at[sl])

        @pl.when(is_last)
        def _():
            pltpu.sync_copy(src.at[tl], dst.at[tl])
    else:
        @pl.when(cond)
        def _():
            pltpu.sync_copy(src.at[sl], dst.at[sl])

        @pl.when(cond & is_last)
        def _():
            pltpu.sync_copy(src.at[tl], dst.at[tl])


def _sc_cnt_body(dst0, dst1, z, ones_h, cnt0, cnt1,
                 dst_v, ones_v, sem_a, sem_b, acc):
    cid = lax.axis_index("c")
    sid = lax.axis_index("s")
    _part_copy(z, acc, sid)
    pltpu.sync_copy(ones_h, ones_v)
    plsc.subcore_barrier()

    def s_start(j, sem):
        pltpu.async_copy(ones_v, acc.at[dst_v.at[j]], sem, add=True)

    def s_wait(j, sem):
        pltpu.make_async_copy(ones_v, acc.at[dst_v.at[j]], sem).wait()

    def pipe(k, carry):
        ja = 2 * k
        jb = 2 * k + 1
        s_wait(ja, sem_a)
        s_start(ja + 2, sem_a)
        s_wait(jb, sem_b)
        s_start(jb + 2, sem_b)
        return carry

    for g in range(NGRP + 1):
        nch = GCH if g < NGRP else GTAIL
        sl_g = pl.ds(g * GCH, nch)
        sl_v = pl.ds(0, nch)

        @pl.when(cid == 0)
        def _():
            pltpu.sync_copy(dst0.at[sid, sl_g], dst_v.at[sl_v])

        @pl.when(cid == 1)
        def _():
            pltpu.sync_copy(dst1.at[sid, sl_g], dst_v.at[sl_v])

        s_start(0, sem_a)
        s_start(1, sem_b)
        lax.fori_loop(0, nch // 2 - 1, pipe, 0)
        if nch % 2:
            s_wait(nch - 3, sem_a)
            s_start(nch - 1, sem_a)
            s_wait(nch - 2, sem_b)
            s_wait(nch - 1, sem_a)
        else:
            s_wait(nch - 2, sem_a)
            s_wait(nch - 1, sem_b)

    plsc.subcore_barrier()
    _part_copy(acc, cnt0, sid, cond=cid == 0)
    _part_copy(acc, cnt1, sid, cond=cid == 1)


_sc_cnt = pl.kernel(
    _sc_cnt_body,
    out_type=(jax.ShapeDtypeStruct((N, CW), _f32),
              jax.ShapeDtypeStruct((N, CW), _f32)),
    mesh=_mesh,
    scratch_types=[
        pltpu.VMEM((GCH, K), jnp.int32),
        pltpu.VMEM((K, CW), _f32),
        pltpu.SemaphoreType.DMA,
        pltpu.SemaphoreType.DMA,
        pltpu.VMEM_SHARED((N, CW), _f32),
    ],
)


def _make_sc_seg(nchunks):
    npass = nchunks // 2

    def body(*refs):
        pos = 0
        hs = refs[pos:pos + nchunks]; pos += nchunks
        srcr, dstr, z = refs[pos:pos + 3]; pos += 3
        outs = refs[pos:pos + nchunks]; pos += nchunks
        (src_v, dst_v, rows_a, rows_b,
         sem_a, sem_b, sem_sa, sem_sb, acc) = refs[pos:pos + 9]

        cid = lax.axis_index("c")
        sid = lax.axis_index("s")

        def pipe_loop(tbl, nch):
            def g_start(j, buf, sem):
                pltpu.async_copy(tbl.at[src_v.at[j]], buf, sem)

            def g_wait(j, buf, sem):
                pltpu.make_async_copy(tbl.at[src_v.at[j]], buf, sem).wait()

            def s_start(j, buf, sem):
                pltpu.async_copy(buf, acc.at[dst_v.at[j]], sem, add=True)

            def s_wait(j, buf, sem):
                pltpu.make_async_copy(buf, acc.at[dst_v.at[j]], sem).wait()

            g_start(0, rows_a, sem_a)
            g_start(1, rows_b, sem_b)

            def body2(k, carry):
                ja = 2 * k
                jb = 2 * k + 1
                jn_a = jnp.minimum(ja + 2, nch - 1)
                jn_b = jnp.minimum(jb + 2, nch - 1)
                g_wait(ja, rows_a, sem_a)
                s_start(ja, rows_a, sem_sa)
                g_wait(jb, rows_b, sem_b)
                s_start(jb, rows_b, sem_sb)
                s_wait(ja, rows_a, sem_sa)
                g_start(jn_a, rows_a, sem_a)
                s_wait(jb, rows_b, sem_sb)
                g_start(jn_b, rows_b, sem_b)
                return carry

            lax.fori_loop(0, nch // 2, body2, 0)
            if nch % 2:
                jl = nch - 1
                g_wait(jl, rows_a, sem_a)
                s_start(jl, rows_a, sem_sa)
                g_wait(jl, rows_b, sem_b)
                s_wait(jl, rows_a, sem_sa)
            else:
                g_wait(nch - 1, rows_a, sem_a)
                g_wait(nch - 1, rows_b, sem_b)

        for p in range(npass):
            h_sc0 = hs[p]
            h_sc1 = hs[npass + p]

            _part_copy(z, acc, sid)
            plsc.subcore_barrier()

            for g in range(NGRP + 1):
                nch = GCH if g < NGRP else GTAIL
                sl_g = pl.ds(g * GCH, nch)
                sl_v = pl.ds(0, nch)
                pltpu.sync_copy(srcr.at[sid, sl_g], src_v.at[sl_v])
                pltpu.sync_copy(dstr.at[sid, sl_g], dst_v.at[sl_v])

                @pl.when(cid == 0)
                def _():
                    pipe_loop(h_sc0, nch)

                @pl.when(cid == 1)
                def _():
                    pipe_loop(h_sc1, nch)

            plsc.subcore_barrier()
            _part_copy(acc, outs[p], sid, cond=cid == 0)
            _part_copy(acc, outs[npass + p], sid, cond=cid == 1)
            plsc.subcore_barrier()

    out_type = [jax.ShapeDtypeStruct((N, CW), _f32) for _ in range(nchunks)]
    scratch = [
        pltpu.VMEM((GCH, K), jnp.int32),
        pltpu.VMEM((GCH, K), jnp.int32),
        pltpu.VMEM((K, CW), _f32),
        pltpu.VMEM((K, CW), _f32),
        pltpu.SemaphoreType.DMA,
        pltpu.SemaphoreType.DMA,
        pltpu.SemaphoreType.DMA,
        pltpu.SemaphoreType.DMA,
        pltpu.VMEM_SHARED((N, CW), _f32),
    ]
    return pl.kernel(body, out_type=tuple(out_type), mesh=_mesh,
                     scratch_types=scratch)


_sc_l0 = _make_sc_seg(NCX)
_sc_agg = _make_sc_seg(NCH)


def _stats_update(st_ref, b, y, i):
    st = jnp.concatenate(
        [jnp.sum(y, axis=0, keepdims=True),
         jnp.sum(y * y, axis=0, keepdims=True),
         jnp.zeros((6, D_H), _f32)], axis=0)

    @pl.when(i == 0)
    def _():
        st_ref[b] = st

    @pl.when(i != 0)
    def _():
        st_ref[b] += st


def _dense_l0_body(*refs):
    x_ref = refs[0]
    aggs = refs[1:1 + 2 * NCX]
    c0_ref, c1_ref, wl_ref, wr_ref, bl_ref = refs[1 + 2 * NCX:6 + 2 * NCX]
    y0_ref, y1_ref, st_ref = refs[6 + 2 * NCX:]
    i = pl.program_id(0)
    xw = jnp.dot(x_ref[...], wr_ref[...], preferred_element_type=_f32)
    for b in range(2):
        ac = aggs[b * NCX:(b + 1) * NCX]
        cref = c0_ref if b == 0 else c1_ref
        inv = 1.0 / jnp.maximum(cref[0], 1.0)
        agg = jnp.concatenate([a[...] for a in ac], axis=1) * inv
        y = jnp.dot(agg, wl_ref[...], preferred_element_type=_f32) \
            + xw + bl_ref[...]
        (y0_ref if b == 0 else y1_ref)[...] = y
        _stats_update(st_ref, b, y, i)


def _dense_l1_body(*refs):
    hs = refs[0:2 * NCH]
    aggs = refs[2 * NCH:4 * NCH]
    c0_ref, c1_ref, wl_ref, wr_ref, bl_ref = refs[4 * NCH:5 + 4 * NCH]
    y0_ref, y1_ref, st_ref = refs[5 + 4 * NCH:]
    i = pl.program_id(0)
    for b in range(2):
        hc = hs[b * NCH:(b + 1) * NCH]
        ac = aggs[b * NCH:(b + 1) * NCH]
        cref = c0_ref if b == 0 else c1_ref
        xcat = jnp.concatenate([h[...] for h in hc], axis=1)
        inv = 1.0 / jnp.maximum(cref[0], 1.0)
        agg = jnp.concatenate([a[...] for a in ac], axis=1) * inv
        y = (jnp.dot(agg, wl_ref[...], preferred_element_type=_f32)
             + jnp.dot(xcat, wr_ref[...], preferred_element_type=_f32)
             + bl_ref[...])
        (y0_ref if b == 0 else y1_ref)[...] = y
        _stats_update(st_ref, b, y, i)


def _bn_body(*refs):
    y0_ref, y1_ref, st_ref, g_ref, bb_ref = refs[:5]
    h_refs = refs[5:]
    for b in range(2):
        st = st_ref[b]
        m = st[0:1, :] * (1.0 / N)
        var = st[1:2, :] * (1.0 / N) - m * m
        s = g_ref[...] * lax.rsqrt(var + 1e-5)
        t = bb_ref[...] - m * s
        y = (y0_ref if b == 0 else y1_ref)[...]
        h = jnp.maximum(y * s + t, 0.0)
        for c in range(NCH):
            h_refs[b * NCH + c][...] = h[:, c * CW:(c + 1) * CW]


def _final_body(*refs):
    aggs = refs[0:2 * NCH]
    c0_ref, c1_ref = refs[2 * NCH:2 + 2 * NCH]
    hs = refs[2 + 2 * NCH:2 + 4 * NCH]
    batch_ref, wl_ref, wr_ref, bl_ref = refs[2 + 4 * NCH:6 + 4 * NCH]
    out_ref, pool_a, pool_h, gcnt = refs[6 + 4 * NCH:]
    i = pl.program_id(0)
    inv0 = 1.0 / jnp.maximum(c0_ref[0], 1.0)
    inv1 = 1.0 / jnp.maximum(c1_ref[0], 1.0)
    agg = (jnp.concatenate([a[...] for a in aggs[:NCH]], axis=1) * inv0
           + jnp.concatenate([a[...] for a in aggs[NCH:]], axis=1) * inv1
           ) * 0.5
    h = (jnp.concatenate([x[...] for x in hs[:NCH]], axis=1)
         + jnp.concatenate([x[...] for x in hs[NCH:]], axis=1)) * 0.5
    bt = batch_ref[0]
    onehot = (bt == lax.broadcasted_iota(jnp.int32, (R, G), 1)).astype(_f32)
    dn = (((0,), (0,)), ((), ()))
    pa = lax.dot_general(onehot, agg, dn, preferred_element_type=_f32)
    ph = lax.dot_general(onehot, h, dn, preferred_element_type=_f32)
    gc = lax.dot_general(onehot, jnp.ones((R, 8), _f32), dn,
                         preferred_element_type=_f32)

    @pl.when(i == 0)
    def _():
        pool_a[...] = pa
        pool_h[...] = ph
        gcnt[...] = gc

    @pl.when(i != 0)
    def _():
        pool_a[...] += pa
        pool_h[...] += ph
        gcnt[...] += gc

    @pl.when(i == NB - 1)
    def _():
        gcv = jnp.maximum(gcnt[...][:, 0:1], 1.0)
        out_ref[...] = (
            jnp.dot(pool_a[...] / gcv, wl_ref[...],
                    preferred_element_type=_f32)
            + jnp.dot(pool_h[...] / gcv, wr_ref[...],
                      preferred_element_type=_f32)
            + bl_ref[...])


def _row_spec(w):
    return pl.BlockSpec((R, w), lambda i: (i, 0))


def _full_spec(r, c):
    return pl.BlockSpec((r, c), lambda i: (0, 0))


_st_spec = pl.BlockSpec((2, 8, D_H), lambda i: (0, 0, 0))
_cnt_spec = pl.BlockSpec((1, R, 1), lambda i: (i, 0, 0))

_dense_l0 = pl.pallas_call(
    _dense_l0_body,
    grid=(NB,),
    in_specs=[_row_spec(D_IN)] + [_row_spec(CW)] * (2 * NCX) + [
        _cnt_spec, _cnt_spec,
        _full_spec(D_IN, D_H), _full_spec(D_IN, D_H), _full_spec(1, D_H),
    ],
    out_specs=[_row_spec(D_H), _row_spec(D_H), _st_spec],
    out_shape=[
        jax.ShapeDtypeStruct((N, D_H), _f32),
        jax.ShapeDtypeStruct((N, D_H), _f32),
        jax.ShapeDtypeStruct((2, 8, D_H), _f32),
    ],
)

_dense_l1 = pl.pallas_call(
    _dense_l1_body,
    grid=(NB,),
    in_specs=[_row_spec(CW)] * (4 * NCH) + [
        _cnt_spec, _cnt_spec,
        _full_spec(D_H, D_H), _full_spec(D_H, D_H), _full_spec(1, D_H),
    ],
    out_specs=[_row_spec(D_H), _row_spec(D_H), _st_spec],
    out_shape=[
        jax.ShapeDtypeStruct((N, D_H), _f32),
        jax.ShapeDtypeStruct((N, D_H), _f32),
        jax.ShapeDtypeStruct((2, 8, D_H), _f32),
    ],
)

_bn_relu = pl.pallas_call(
    _bn_body,
    grid=(NB,),
    in_specs=[
        _row_spec(D_H), _row_spec(D_H), _st_spec,
        _full_spec(1, D_H), _full_spec(1, D_H),
    ],
    out_specs=[_row_spec(CW)] * (2 * NCH),
    out_shape=[jax.ShapeDtypeStruct((N, CW), _f32)] * (2 * NCH),
)

_final = pl.pallas_call(
    _final_body,
    grid=(NB,),
    in_specs=[_row_spec(CW)] * (2 * NCH) + [
        _cnt_spec, _cnt_spec,
    ] + [_row_spec(CW)] * (2 * NCH) + [
        pl.BlockSpec((1, R, 1), lambda i: (i, 0, 0)),
        _full_spec(D_H, D_H), _full_spec(D_H, D_H), _full_spec(1, D_H),
    ],
    out_specs=pl.BlockSpec((G, D_H), lambda i: (0, 0)),
    out_shape=jax.ShapeDtypeStruct((G, D_H), _f32),
    scratch_shapes=[
        pltpu.VMEM((G, D_H), _f32),
        pltpu.VMEM((G, D_H), _f32),
        pltpu.VMEM((G, 8), _f32),
    ],
)


def kernel(x, edge_index_sc, edge_index_fc, batch,
           Wl0, bl0, Wr0, g0, b0,
           Wl1, bl1, Wr1, g1, b1,
           Wl2, bl2, Wr2):
    xc = [x[:, c * CW:(c + 1) * CW] for c in range(NCX)]
    z = jnp.zeros((N, CW), _f32)

    def edges(ei):
        return (ei[0].reshape(NT, NCHUNK, K), ei[1].reshape(NT, NCHUNK, K))

    src0, dst0 = edges(edge_index_sc)
    src1, dst1 = edges(edge_index_fc)

    ones_rows = jnp.ones((K, CW), _f32)
    cnt0_t, cnt1_t = _sc_cnt(dst0, dst1, z, ones_rows)
    cnt0 = cnt0_t[:, 0].reshape(NB, R, 1)
    cnt1 = cnt1_t[:, 0].reshape(NB, R, 1)

    a0_0 = _sc_l0(*xc, src0, dst0, z)
    a0_1 = _sc_l0(*xc, src1, dst1, z)

    y00, y01, st0 = _dense_l0(x, *a0_0, *a0_1, cnt0, cnt1,
                              Wl0, Wr0, bl0.reshape(1, D_H))
    h0 = _bn_relu(y00, y01, st0, g0.reshape(1, D_H), b0.reshape(1, D_H))

    agg1_0 = _sc_agg(*h0[:NCH], src0, dst0, z)
    agg1_1 = _sc_agg(*h0[NCH:], src1, dst1, z)

    y10, y11, st1 = _dense_l1(*h0, *agg1_0, *agg1_1, cnt0, cnt1,
                              Wl1, Wr1, bl1.reshape(1, D_H))
    h1 = _bn_relu(y10, y11, st1, g1.reshape(1, D_H), b1.reshape(1, D_H))

    agg2_0 = _sc_agg(*h1[:NCH], src0, dst0, z)
    agg2_1 = _sc_agg(*h1[NCH:], src1, dst1, z)

    out = _final(*agg2_0, *agg2_1, cnt0, cnt1, *h1,
                 batch.reshape(NB, R, 1), Wl2, Wr2, bl2.reshape(1, D_H))
    return out
